# Initial kernel scaffold; baseline (speedup 1.0000x reference)
import jax
import jax.numpy as jnp
from jax.experimental import pallas as pl


def kernel(x_enc, x_mark_enc, y_batch, x_mark_dec, tok_w, mark_w, qk_w, v_w, out_w, out_b, n1g, n1b, f1w, f1b, f2w, f2b, n2g, n2b, ng, nb, proj_w, proj_b):
    raise NotImplementedError("write your pallas kernel here")


# speedup vs baseline: 63.2265x; 63.2265x over previous
import functools
import numpy as np
import jax
import jax.numpy as jnp
from jax.experimental import pallas as pl

B = 2; SEQ = 2048; PRED = 2048; LTOT = SEQ + PRED
D_FEAT = 7; C_OUT = 7; D_MARK = 4
D = 256; H = 8; DH = D // H; E_LAYERS = 3; D_FF = 256
BUCKET = 64; N_HASHES = 4; N_BUCKETS = LTOT // BUCKET
NCH = N_HASHES * LTOT // BUCKET
L4 = N_HASHES * LTOT

_pos = np.arange(LTOT)[:, None].astype(np.float64)
_div = np.exp(np.arange(0, D, 2).astype(np.float64) * -(np.log(10000.0) / D))
_pe = np.zeros((LTOT, D)); _pe[:, 0::2] = np.sin(_pos * _div); _pe[:, 1::2] = np.cos(_pos * _div)
_PE = jnp.asarray(_pe, jnp.float32)
_ROT = jnp.asarray(np.stack([np.random.RandomState(100 + i).randn(DH, N_HASHES, N_BUCKETS // 2) for i in range(E_LAYERS)]), jnp.float32)

_INTERPRET = False


def _pcall(body, **kw):
    return pl.pallas_call(body, interpret=_INTERPRET, **kw)


def _embed_body(xe_ref, xm_ref, pe_ref, w_ref, mw_ref, o_ref):
    xe = xe_ref[0]
    prev = jnp.concatenate([xe[-1:], xe[:-1]], axis=0)
    nxt = jnp.concatenate([xe[1:], xe[:1]], axis=0)
    w = w_ref[...]
    y = prev @ w[0] + xe @ w[1] + nxt @ w[2]
    o_ref[0] = y + pe_ref[...] + xm_ref[0] @ mw_ref[...]


def _embed(xe, xm, tok_w, mark_w):
    w = jnp.transpose(tok_w, (2, 1, 0))
    return _pcall(
        _embed_body,
        grid=(B,),
        in_specs=[
            pl.BlockSpec((1, LTOT, D_FEAT), lambda b: (b, 0, 0)),
            pl.BlockSpec((1, LTOT, D_MARK), lambda b: (b, 0, 0)),
            pl.BlockSpec((LTOT, D), lambda b: (0, 0)),
            pl.BlockSpec((3, D_FEAT, D), lambda b: (0, 0, 0)),
            pl.BlockSpec((D_MARK, D), lambda b: (0, 0)),
        ],
        out_specs=pl.BlockSpec((1, LTOT, D), lambda b: (b, 0, 0)),
        out_shape=jax.ShapeDtypeStruct((B, LTOT, D), jnp.float32),
    )(xe, xm, _PE, w, mark_w)


def _prep_body(x_ref, qkw_ref, vw_ref, rot_ref, qk_ref, v_ref, dest_ref):
    x = x_ref[0]
    qk = x @ qkw_ref[0]
    qk_ref[0] = qk
    v_ref[0] = x @ vw_ref[0]
    rotated = qk @ rot_ref[...]
    nb2 = N_BUCKETS // 2
    iota64 = jax.lax.broadcasted_iota(jnp.int32, (1, N_BUCKETS), 1).astype(jnp.float32)
    CH = 512
    r0 = jax.lax.broadcasted_iota(jnp.int32, (CH, CH), 0)
    c0 = jax.lax.broadcasted_iota(jnp.int32, (CH, CH), 1)
    tril = (c0 <= r0).astype(jnp.float32)
    ru = jax.lax.broadcasted_iota(jnp.int32, (N_BUCKETS, N_BUCKETS), 0)
    cu = jax.lax.broadcasted_iota(jnp.int32, (N_BUCKETS, N_BUCKETS), 1)
    sut = (ru < cu).astype(jnp.float32)
    for n in range(N_HASHES):
        z = rotated[:, n * nb2:(n + 1) * nb2]
        cat = jnp.concatenate([z, -z], axis=1)
        m = jnp.max(cat, axis=1, keepdims=True)
        c = jnp.min(jnp.where(cat == m, iota64, 1e9), axis=1, keepdims=True)
        oh = (c == iota64).astype(jnp.float32)
        carry = jnp.zeros((1, N_BUCKETS), jnp.float32)
        cums = []
        for k in range(LTOT // CH):
            blk = oh[k * CH:(k + 1) * CH]
            cums.append(tril @ blk + carry)
            carry = cums[-1][-1:, :]
        cum = jnp.concatenate(cums, axis=0)
        rank = jnp.sum(oh * cum, axis=1, keepdims=True) - 1.0
        off = carry @ sut
        base = jnp.sum(oh * off, axis=1, keepdims=True)
        dest_ref[0, :, n:n + 1] = (base + rank + float(n * LTOT)).astype(jnp.int32)


def _prep(x, qk_w, v_w, rot2):
    f = _pcall(
        _prep_body,
        grid=(B, H),
        in_specs=[
            pl.BlockSpec((1, LTOT, D), lambda b, h: (b, 0, 0)),
            pl.BlockSpec((1, D, DH), lambda b, h: (h, 0, 0)),
            pl.BlockSpec((1, D, DH), lambda b, h: (h, 0, 0)),
            pl.BlockSpec((DH, N_HASHES * (N_BUCKETS // 2)), lambda b, h: (0, 0)),
        ],
        out_specs=[
            pl.BlockSpec((1, LTOT, DH), lambda b, h: (b * H + h, 0, 0)),
            pl.BlockSpec((1, LTOT, DH), lambda b, h: (b * H + h, 0, 0)),
            pl.BlockSpec((1, LTOT, N_HASHES), lambda b, h: (b * H + h, 0, 0)),
        ],
        out_shape=[
            jax.ShapeDtypeStruct((B * H, LTOT, DH), jnp.float32),
            jax.ShapeDtypeStruct((B * H, LTOT, DH), jnp.float32),
            jax.ShapeDtypeStruct((B * H, LTOT, N_HASHES), jnp.int32),
        ],
    )(x, qk_w, v_w, rot2)
    return f


SPAN = 2048
NSPAN = L4 // SPAN
CPS = SPAN // BUCKET


def _attn_body(cur_ref, prv_ref, stkv_ref, so_ref):
    def compute(ci, kprev):
        q0 = cur_ref[0, pl.ds(ci * BUCKET, BUCKET), :]
        bq = q0[:, 0:DH]
        tq = q0[:, 64:65]
        kv = jnp.concatenate([q0, kprev], axis=0)
        bk = kv[:, 0:DH]
        bv = kv[:, DH:2 * DH]
        nrm = jnp.sqrt(jnp.sum(bk * bk, axis=1, keepdims=True))
        bkn = bk / jnp.maximum(nrm, 1e-12)
        dots = jax.lax.dot_general(bq, bkn, (((1,), (1,)), ((), ()))) * (DH ** -0.5)
        tkv = stkv_ref[0, pl.ds(ci, 1), :]
        dots = jnp.where(tq == tkv, -5e4, dots)
        m = jnp.max(dots, axis=1, keepdims=True)
        p = jnp.exp(dots - m)
        s = jnp.sum(p, axis=1, keepdims=True)
        lse = m + jnp.log(s)
        bo = (p / s) @ bv
        out = jnp.concatenate([bo, lse, jnp.zeros((BUCKET, 128 - DH - 1), jnp.float32)], axis=1)
        so_ref[0, pl.ds(ci * BUCKET, BUCKET), :] = out

    compute(0, prv_ref[0, pl.ds(SPAN - BUCKET, BUCKET), :])

    def loop_body(ci, _):
        compute(ci, cur_ref[0, pl.ds((ci - 1) * BUCKET, BUCKET), :])
        return 0

    jax.lax.fori_loop(1, CPS, loop_body, 0)


def _attn(qkvp, stkv):
    return _pcall(
        _attn_body,
        grid=(B * H, NSPAN),
        in_specs=[
            pl.BlockSpec((1, SPAN, 128), lambda g, s: (g, s, 0)),
            pl.BlockSpec((1, SPAN, 128), lambda g, s: (g, (s + NSPAN - 1) % NSPAN, 0)),
            pl.BlockSpec((1, CPS, 128), lambda g, s: (g, s, 0)),
        ],
        out_specs=pl.BlockSpec((1, SPAN, 128), lambda g, s: (g, s, 0)),
        out_shape=jax.ShapeDtypeStruct((B * H, L4, 128), jnp.float32),
    )(qkvp, qkvp, stkv)


SPAN2 = 2048


def _combine_body(ou_ref, o_ref):
    rows = [ou_ref[0, n] for n in range(N_HASHES)]
    ls = [r[:, DH:DH + 1] for r in rows]
    m = ls[0]
    for n in range(1, N_HASHES):
        m = jnp.maximum(m, ls[n])
    ps = [jnp.exp(l - m) for l in ls]
    z = ps[0]
    for n in range(1, N_HASHES):
        z = z + ps[n]
    acc = rows[0][:, 0:DH] * ps[0]
    for n in range(1, N_HASHES):
        acc = acc + rows[n][:, 0:DH] * ps[n]
    o_ref[0] = acc / z


def _combine(ou4):
    return _pcall(
        _combine_body,
        grid=(B * H, LTOT // SPAN2),
        in_specs=[pl.BlockSpec((1, N_HASHES, SPAN2, 128), lambda g, s: (g, 0, s, 0))],
        out_specs=pl.BlockSpec((1, SPAN2, DH), lambda g, s: (g, s, 0)),
        out_shape=jax.ShapeDtypeStruct((B * H, LTOT, DH), jnp.float32),
    )(ou4)


def _erf(x):
    a1, a2, a3, a4, a5 = 0.254829592, -0.284496736, 1.421413741, -1.453152027, 1.061405429
    p = 0.3275911
    s = jnp.sign(x)
    ax = jnp.abs(x)
    t = 1.0 / (1.0 + p * ax)
    y = 1.0 - (((((a5 * t + a4) * t) + a3) * t + a2) * t + a1) * t * jnp.exp(-ax * ax)
    return s * y


def _ln_in(x, g, b):
    m = jnp.mean(x, axis=-1, keepdims=True)
    v = jnp.mean((x - m) ** 2, axis=-1, keepdims=True)
    return (x - m) / jnp.sqrt(v + 1e-5) * g + b


def _post_body(a_ref, x_ref, ow_ref, ob_ref, n1g_ref, n1b_ref, f1w_ref, f1b_ref,
               f2w_ref, f2b_ref, n2g_ref, n2b_ref, o_ref):
    a = a_ref[0] @ ow_ref[...] + ob_ref[...]
    x = _ln_in(x_ref[0] + a, n1g_ref[...], n1b_ref[...])
    h1 = x @ f1w_ref[...] + f1b_ref[...]
    g = h1 * 0.5 * (1.0 + _erf(h1 * (2.0 ** -0.5)))
    y = g @ f2w_ref[...] + f2b_ref[...]
    o_ref[0] = _ln_in(x + y, n2g_ref[...], n2b_ref[...])


def _post(attn, x, ow, ob, n1g, n1b, f1w, f1b, f2w, f2b, n2g, n2b):
    return _pcall(
        _post_body,
        grid=(B,),
        in_specs=[
            pl.BlockSpec((1, LTOT, D), lambda i: (i, 0, 0)),
            pl.BlockSpec((1, LTOT, D), lambda i: (i, 0, 0)),
            pl.BlockSpec((D, D), lambda i: (0, 0)),
            pl.BlockSpec((1, D), lambda i: (0, 0)),
            pl.BlockSpec((1, D), lambda i: (0, 0)),
            pl.BlockSpec((1, D), lambda i: (0, 0)),
            pl.BlockSpec((D, D_FF), lambda i: (0, 0)),
            pl.BlockSpec((1, D_FF), lambda i: (0, 0)),
            pl.BlockSpec((D_FF, D), lambda i: (0, 0)),
            pl.BlockSpec((1, D), lambda i: (0, 0)),
            pl.BlockSpec((1, D), lambda i: (0, 0)),
            pl.BlockSpec((1, D), lambda i: (0, 0)),
        ],
        out_specs=pl.BlockSpec((1, LTOT, D), lambda i: (i, 0, 0)),
        out_shape=jax.ShapeDtypeStruct((B, LTOT, D), jnp.float32),
    )(attn, x, ow, ob.reshape(1, D), n1g.reshape(1, D), n1b.reshape(1, D),
      f1w, f1b.reshape(1, D_FF), f2w, f2b.reshape(1, D), n2g.reshape(1, D), n2b.reshape(1, D))


def _final_body(x_ref, ng_ref, nb_ref, pw_ref, pb_ref, o_ref):
    x = _ln_in(x_ref[0, pl.ds(LTOT - PRED, PRED), :], ng_ref[...], nb_ref[...])
    o_ref[0] = x @ pw_ref[...] + pb_ref[...]


def _final(x, ng, nb, pw, pb):
    return _pcall(
        _final_body,
        grid=(B,),
        in_specs=[
            pl.BlockSpec((1, LTOT, D), lambda i: (i, 0, 0)),
            pl.BlockSpec((1, D), lambda i: (0, 0)),
            pl.BlockSpec((1, D), lambda i: (0, 0)),
            pl.BlockSpec((D, C_OUT), lambda i: (0, 0)),
            pl.BlockSpec((1, C_OUT), lambda i: (0, 0)),
        ],
        out_specs=pl.BlockSpec((1, PRED, C_OUT), lambda i: (i, 0, 0)),
        out_shape=jax.ShapeDtypeStruct((B, PRED, C_OUT), jnp.float32),
    )(x, ng.reshape(1, D), nb.reshape(1, D), pw, pb.reshape(1, C_OUT))


def _apply_sort(qk, v, dest):
    g = B * H
    destg = jnp.transpose(dest, (0, 2, 1)).reshape(g, L4)
    posf = jnp.arange(LTOT, dtype=jnp.float32)
    row = jnp.concatenate([
        qk, v, jnp.zeros((g, LTOT, 64 - 2 * DH), jnp.float32),
        posf[None, :, None] * jnp.ones((g, 1, 1), jnp.float32),
        jnp.zeros((g, LTOT, 63), jnp.float32)], axis=-1)
    src = jnp.tile(row, (1, N_HASHES, 1)).reshape(g * L4, 128)
    flat = (jnp.arange(g, dtype=jnp.int32)[:, None] * L4 + destg).reshape(-1)
    qkvp = jnp.zeros((g * L4, 128), jnp.float32).at[flat].set(src).reshape(g, L4, 128)
    st = qkvp[:, :, 64]
    return qkvp, st, destg


def _apply_unsort(so, destg):
    return jnp.take_along_axis(so, destg[:, :, None], axis=1)


def kernel(x_enc, x_mark_enc, y_batch, x_mark_dec, tok_w, mark_w, qk_w, v_w, out_w, out_b,
           n1g, n1b, f1w, f1b, f2w, f2b, n2g, n2b, ng, nb, proj_w, proj_b):
    xe = jnp.concatenate([x_enc, jnp.zeros((B, PRED, D_FEAT), jnp.float32)], axis=1)
    xm = jnp.concatenate([x_mark_enc, x_mark_dec], axis=1)
    x = _embed(xe, xm, tok_w, mark_w)
    for i in range(E_LAYERS):
        rot2 = _ROT[i].reshape(DH, N_HASHES * (N_BUCKETS // 2))
        qkwh = jnp.transpose(qk_w[i].reshape(D, H, DH), (1, 0, 2))
        vwh = jnp.transpose(v_w[i].reshape(D, H, DH), (1, 0, 2))
        qk, v, dest = _prep(x, qkwh, vwh, rot2)
        qkvp, st, destg = _apply_sort(qk, v, dest)
        stc = st.reshape(B * H, NCH, BUCKET)
        stkv = jnp.concatenate([stc, jnp.roll(stc, 1, axis=1)], axis=2)
        so = _attn(qkvp, stkv)
        ou4 = _apply_unsort(so, destg).reshape(B * H, N_HASHES, LTOT, 128)
        a = jnp.transpose(_combine(ou4).reshape(B, H, LTOT, DH), (0, 2, 1, 3)).reshape(B, LTOT, D)
        x = _post(a, x, out_w[i], out_b[i], n1g[i], n1b[i], f1w[i], f1b[i],
                  f2w[i], f2b[i], n2g[i], n2b[i])
    return _final(x, ng, nb, proj_w, proj_b)


# speedup vs baseline: 112.2663x; 1.7756x over previous
import functools
import numpy as np
import jax
import jax.numpy as jnp
from jax.experimental import pallas as pl
from jax.experimental.pallas import tpu as pltpu
from jax.experimental.pallas import tpu_sc as plsc

B = 2; SEQ = 2048; PRED = 2048; LTOT = SEQ + PRED
D_FEAT = 7; C_OUT = 7; D_MARK = 4
D = 256; H = 8; DH = D // H; E_LAYERS = 3; D_FF = 256
BUCKET = 64; N_HASHES = 4; N_BUCKETS = LTOT // BUCKET
NCH = N_HASHES * LTOT // BUCKET
L4 = N_HASHES * LTOT

_pos = np.arange(LTOT)[:, None].astype(np.float64)
_div = np.exp(np.arange(0, D, 2).astype(np.float64) * -(np.log(10000.0) / D))
_pe = np.zeros((LTOT, D)); _pe[:, 0::2] = np.sin(_pos * _div); _pe[:, 1::2] = np.cos(_pos * _div)
_PE = jnp.asarray(_pe, jnp.float32)
_ROT = jnp.asarray(np.stack([np.random.RandomState(100 + i).randn(DH, N_HASHES, N_BUCKETS // 2) for i in range(E_LAYERS)]), jnp.float32)

_INTERPRET = False


def _pcall(body, **kw):
    return pl.pallas_call(body, interpret=_INTERPRET, **kw)


def _embed_body(xe_ref, xm_ref, pe_ref, w_ref, mw_ref, o_ref):
    xe = xe_ref[0]
    prev = jnp.concatenate([xe[-1:], xe[:-1]], axis=0)
    nxt = jnp.concatenate([xe[1:], xe[:1]], axis=0)
    w = w_ref[...]
    y = prev @ w[0] + xe @ w[1] + nxt @ w[2]
    o_ref[0] = y + pe_ref[...] + xm_ref[0] @ mw_ref[...]


def _embed(xe, xm, tok_w, mark_w):
    w = jnp.transpose(tok_w, (2, 1, 0))
    return _pcall(
        _embed_body,
        grid=(B,),
        in_specs=[
            pl.BlockSpec((1, LTOT, D_FEAT), lambda b: (b, 0, 0)),
            pl.BlockSpec((1, LTOT, D_MARK), lambda b: (b, 0, 0)),
            pl.BlockSpec((LTOT, D), lambda b: (0, 0)),
            pl.BlockSpec((3, D_FEAT, D), lambda b: (0, 0, 0)),
            pl.BlockSpec((D_MARK, D), lambda b: (0, 0)),
        ],
        out_specs=pl.BlockSpec((1, LTOT, D), lambda b: (b, 0, 0)),
        out_shape=jax.ShapeDtypeStruct((B, LTOT, D), jnp.float32),
    )(xe, xm, _PE, w, mark_w)


def _prep_body(x_ref, qkw_ref, vw_ref, rot_ref, row_ref, dest_ref):
    x = x_ref[0]
    qk = x @ qkw_ref[0]
    v = x @ vw_ref[0]
    posc = jax.lax.broadcasted_iota(jnp.int32, (LTOT, 1), 0).astype(jnp.float32)
    row_ref[0] = jnp.concatenate([
        qk, v, posc, jnp.zeros((LTOT, 63), jnp.float32)], axis=1)
    rotated = qk @ rot_ref[...]
    nb2 = N_BUCKETS // 2
    iota64 = jax.lax.broadcasted_iota(jnp.int32, (1, N_BUCKETS), 1).astype(jnp.float32)
    CH = 512
    r0 = jax.lax.broadcasted_iota(jnp.int32, (CH, CH), 0)
    c0 = jax.lax.broadcasted_iota(jnp.int32, (CH, CH), 1)
    tril = (c0 <= r0).astype(jnp.float32)
    ru = jax.lax.broadcasted_iota(jnp.int32, (N_BUCKETS, N_BUCKETS), 0)
    cu = jax.lax.broadcasted_iota(jnp.int32, (N_BUCKETS, N_BUCKETS), 1)
    sut = (ru < cu).astype(jnp.float32)
    for n in range(N_HASHES):
        z = rotated[:, n * nb2:(n + 1) * nb2]
        cat = jnp.concatenate([z, -z], axis=1)
        m = jnp.max(cat, axis=1, keepdims=True)
        c = jnp.min(jnp.where(cat == m, iota64, 1e9), axis=1, keepdims=True)
        oh = (c == iota64).astype(jnp.float32)
        carry = jnp.zeros((1, N_BUCKETS), jnp.float32)
        cums = []
        for k in range(LTOT // CH):
            blk = oh[k * CH:(k + 1) * CH]
            cums.append(tril @ blk + carry)
            carry = cums[-1][-1:, :]
        cum = jnp.concatenate(cums, axis=0)
        rank = jnp.sum(oh * cum, axis=1, keepdims=True) - 1.0
        off = carry @ sut
        base = jnp.sum(oh * off, axis=1, keepdims=True)
        dest_ref[0, :, n:n + 1] = (base + rank + float(n * LTOT)).astype(jnp.int32)


def _prep(x, qk_w, v_w, rot2):
    f = _pcall(
        _prep_body,
        grid=(B, H),
        in_specs=[
            pl.BlockSpec((1, LTOT, D), lambda b, h: (b, 0, 0)),
            pl.BlockSpec((1, D, DH), lambda b, h: (h, 0, 0)),
            pl.BlockSpec((1, D, DH), lambda b, h: (h, 0, 0)),
            pl.BlockSpec((DH, N_HASHES * (N_BUCKETS // 2)), lambda b, h: (0, 0)),
        ],
        out_specs=[
            pl.BlockSpec((1, LTOT, 128), lambda b, h: (b * H + h, 0, 0)),
            pl.BlockSpec((1, LTOT, N_HASHES), lambda b, h: (b * H + h, 0, 0)),
        ],
        out_shape=[
            jax.ShapeDtypeStruct((B * H, LTOT, 128), jnp.float32),
            jax.ShapeDtypeStruct((B * H, LTOT, N_HASHES), jnp.int32),
        ],
    )(x, qk_w, v_w, rot2)
    return f


SPAN = 2048
NSPAN = L4 // SPAN
CPS = SPAN // BUCKET


def _attn_body(cur_ref, prv_ref, stkv_ref, so_ref):
    def compute(ci, kprev):
        q0 = cur_ref[0, pl.ds(ci * BUCKET, BUCKET), :]
        bq = q0[:, 0:DH]
        tq = q0[:, 64:65]
        kv = jnp.concatenate([q0, kprev], axis=0)
        bk = kv[:, 0:DH]
        bv = kv[:, DH:2 * DH]
        nrm = jnp.sqrt(jnp.sum(bk * bk, axis=1, keepdims=True))
        bkn = bk / jnp.maximum(nrm, 1e-12)
        dots = jax.lax.dot_general(bq, bkn, (((1,), (1,)), ((), ()))) * (DH ** -0.5)
        tkv = stkv_ref[0, pl.ds(ci, 1), :]
        dots = jnp.where(tq == tkv, -5e4, dots)
        m = jnp.max(dots, axis=1, keepdims=True)
        p = jnp.exp(dots - m)
        s = jnp.sum(p, axis=1, keepdims=True)
        lse = m + jnp.log(s)
        bo = (p / s) @ bv
        out = jnp.concatenate([bo, lse, jnp.zeros((BUCKET, 128 - DH - 1), jnp.float32)], axis=1)
        so_ref[0, pl.ds(ci * BUCKET, BUCKET), :] = out

    compute(0, prv_ref[0, pl.ds(SPAN - BUCKET, BUCKET), :])

    def loop_body(ci, _):
        compute(ci, cur_ref[0, pl.ds((ci - 1) * BUCKET, BUCKET), :])
        return 0

    jax.lax.fori_loop(1, CPS, loop_body, 0)


def _attn(qkvp, stkv):
    return _pcall(
        _attn_body,
        grid=(B * H, NSPAN),
        in_specs=[
            pl.BlockSpec((1, SPAN, 128), lambda g, s: (g, s, 0)),
            pl.BlockSpec((1, SPAN, 128), lambda g, s: (g, (s + NSPAN - 1) % NSPAN, 0)),
            pl.BlockSpec((1, CPS, 128), lambda g, s: (g, s, 0)),
        ],
        out_specs=pl.BlockSpec((1, SPAN, 128), lambda g, s: (g, s, 0)),
        out_shape=jax.ShapeDtypeStruct((B * H, L4, 128), jnp.float32),
    )(qkvp, qkvp, stkv)


SPAN2 = 2048


def _combine_body(ou_ref, o_ref):
    rows = [ou_ref[0, n] for n in range(N_HASHES)]
    ls = [r[:, DH:DH + 1] for r in rows]
    m = ls[0]
    for n in range(1, N_HASHES):
        m = jnp.maximum(m, ls[n])
    ps = [jnp.exp(l - m) for l in ls]
    z = ps[0]
    for n in range(1, N_HASHES):
        z = z + ps[n]
    acc = rows[0][:, 0:DH] * ps[0]
    for n in range(1, N_HASHES):
        acc = acc + rows[n][:, 0:DH] * ps[n]
    o_ref[0] = acc / z


def _combine(ou4):
    return _pcall(
        _combine_body,
        grid=(B * H, LTOT // SPAN2),
        in_specs=[pl.BlockSpec((1, N_HASHES, SPAN2, 128), lambda g, s: (g, 0, s, 0))],
        out_specs=pl.BlockSpec((1, SPAN2, DH), lambda g, s: (g, s, 0)),
        out_shape=jax.ShapeDtypeStruct((B * H, LTOT, DH), jnp.float32),
    )(ou4)


def _erf(x):
    a1, a2, a3, a4, a5 = 0.254829592, -0.284496736, 1.421413741, -1.453152027, 1.061405429
    p = 0.3275911
    s = jnp.sign(x)
    ax = jnp.abs(x)
    t = 1.0 / (1.0 + p * ax)
    y = 1.0 - (((((a5 * t + a4) * t) + a3) * t + a2) * t + a1) * t * jnp.exp(-ax * ax)
    return s * y


def _ln_in(x, g, b):
    m = jnp.mean(x, axis=-1, keepdims=True)
    v = jnp.mean((x - m) ** 2, axis=-1, keepdims=True)
    return (x - m) / jnp.sqrt(v + 1e-5) * g + b


def _post_body(a_ref, x_ref, ow_ref, ob_ref, n1g_ref, n1b_ref, f1w_ref, f1b_ref,
               f2w_ref, f2b_ref, n2g_ref, n2b_ref, o_ref):
    a = a_ref[0] @ ow_ref[...] + ob_ref[...]
    x = _ln_in(x_ref[0] + a, n1g_ref[...], n1b_ref[...])
    h1 = x @ f1w_ref[...] + f1b_ref[...]
    g = h1 * 0.5 * (1.0 + _erf(h1 * (2.0 ** -0.5)))
    y = g @ f2w_ref[...] + f2b_ref[...]
    o_ref[0] = _ln_in(x + y, n2g_ref[...], n2b_ref[...])


def _post(attn, x, ow, ob, n1g, n1b, f1w, f1b, f2w, f2b, n2g, n2b):
    return _pcall(
        _post_body,
        grid=(B,),
        in_specs=[
            pl.BlockSpec((1, LTOT, D), lambda i: (i, 0, 0)),
            pl.BlockSpec((1, LTOT, D), lambda i: (i, 0, 0)),
            pl.BlockSpec((D, D), lambda i: (0, 0)),
            pl.BlockSpec((1, D), lambda i: (0, 0)),
            pl.BlockSpec((1, D), lambda i: (0, 0)),
            pl.BlockSpec((1, D), lambda i: (0, 0)),
            pl.BlockSpec((D, D_FF), lambda i: (0, 0)),
            pl.BlockSpec((1, D_FF), lambda i: (0, 0)),
            pl.BlockSpec((D_FF, D), lambda i: (0, 0)),
            pl.BlockSpec((1, D), lambda i: (0, 0)),
            pl.BlockSpec((1, D), lambda i: (0, 0)),
            pl.BlockSpec((1, D), lambda i: (0, 0)),
        ],
        out_specs=pl.BlockSpec((1, LTOT, D), lambda i: (i, 0, 0)),
        out_shape=jax.ShapeDtypeStruct((B, LTOT, D), jnp.float32),
    )(attn, x, ow, ob.reshape(1, D), n1g.reshape(1, D), n1b.reshape(1, D),
      f1w, f1b.reshape(1, D_FF), f2w, f2b.reshape(1, D), n2g.reshape(1, D), n2b.reshape(1, D))


def _final_body(x_ref, ng_ref, nb_ref, pw_ref, pb_ref, o_ref):
    x = _ln_in(x_ref[0, pl.ds(LTOT - PRED, PRED), :], ng_ref[...], nb_ref[...])
    o_ref[0] = x @ pw_ref[...] + pb_ref[...]


def _final(x, ng, nb, pw, pb):
    return _pcall(
        _final_body,
        grid=(B,),
        in_specs=[
            pl.BlockSpec((1, LTOT, D), lambda i: (i, 0, 0)),
            pl.BlockSpec((1, D), lambda i: (0, 0)),
            pl.BlockSpec((1, D), lambda i: (0, 0)),
            pl.BlockSpec((D, C_OUT), lambda i: (0, 0)),
            pl.BlockSpec((1, C_OUT), lambda i: (0, 0)),
        ],
        out_specs=pl.BlockSpec((1, PRED, C_OUT), lambda i: (i, 0, 0)),
        out_shape=jax.ShapeDtypeStruct((B, PRED, C_OUT), jnp.float32),
    )(x, ng.reshape(1, D), nb.reshape(1, D), pw, pb.reshape(1, C_OUT))


SC_CH = 128
SC_NW = 32
_ROWS_PW = L4 // 2


def _sc_worker(kind):
    mesh = plsc.VectorSubcoreMesh(core_axis_name="c", subcore_axis_name="s")

    @functools.partial(
        pl.kernel, mesh=mesh,
        out_type=jax.ShapeDtypeStruct((B * H * L4, 128), jnp.float32),
        scratch_types=[
            pltpu.VMEM((SC_CH,), jnp.int32),
            pltpu.VMEM((SC_CH, 128), jnp.float32),
            pltpu.SemaphoreType.DMA,
        ],
    )
    def k(src_hbm, idx_hbm, out_hbm, idx_v, buf_v, sem):
        wid = jax.lax.axis_index("s") * 2 + jax.lax.axis_index("c")
        g = wid // 2
        half = wid % 2
        for kk in range(_ROWS_PW // SC_CH):
            off = half * _ROWS_PW + kk * SC_CH
            pltpu.sync_copy(idx_hbm.at[g, pl.ds(off, SC_CH)], idx_v)
            if kind == "scatter":
                srow = g * LTOT + (half * _ROWS_PW + kk * SC_CH) % LTOT
                pltpu.sync_copy(src_hbm.at[pl.ds(srow, SC_CH), :], buf_v)
                pltpu.async_copy(buf_v, out_hbm.at[idx_v], sem).wait()
            else:
                pltpu.async_copy(src_hbm.at[idx_v], buf_v, sem).wait()
                pltpu.sync_copy(buf_v, out_hbm.at[pl.ds(g * L4 + off, SC_CH), :])

    return k


_sc_scatter = _sc_worker("scatter")
_sc_gather = _sc_worker("gather")


def _apply_sort(rows, dest):
    g = B * H
    destg = jnp.transpose(dest, (0, 2, 1)).reshape(g, L4)
    idxg = destg + jnp.arange(g, dtype=jnp.int32)[:, None] * L4
    qkvp = _sc_scatter(rows.reshape(g * LTOT, 128), idxg).reshape(g, L4, 128)
    return qkvp, qkvp[:, :, 64], idxg


def _apply_unsort(so, idxg):
    g = B * H
    return _sc_gather(so.reshape(g * L4, 128), idxg).reshape(g, L4, 128)


def kernel(x_enc, x_mark_enc, y_batch, x_mark_dec, tok_w, mark_w, qk_w, v_w, out_w, out_b,
           n1g, n1b, f1w, f1b, f2w, f2b, n2g, n2b, ng, nb, proj_w, proj_b):
    xe = jnp.concatenate([x_enc, jnp.zeros((B, PRED, D_FEAT), jnp.float32)], axis=1)
    xm = jnp.concatenate([x_mark_enc, x_mark_dec], axis=1)
    x = _embed(xe, xm, tok_w, mark_w)
    for i in range(E_LAYERS):
        rot2 = _ROT[i].reshape(DH, N_HASHES * (N_BUCKETS // 2))
        qkwh = jnp.transpose(qk_w[i].reshape(D, H, DH), (1, 0, 2))
        vwh = jnp.transpose(v_w[i].reshape(D, H, DH), (1, 0, 2))
        rows, dest = _prep(x, qkwh, vwh, rot2)
        qkvp, st, idxg = _apply_sort(rows, dest)
        stc = st.reshape(B * H, NCH, BUCKET)
        stkv = jnp.concatenate([stc, jnp.roll(stc, 1, axis=1)], axis=2)
        so = _attn(qkvp, stkv)
        ou4 = _apply_unsort(so, idxg).reshape(B * H, N_HASHES, LTOT, 128)
        a = jnp.transpose(_combine(ou4).reshape(B, H, LTOT, DH), (0, 2, 1, 3)).reshape(B, LTOT, D)
        x = _post(a, x, out_w[i], out_b[i], n1g[i], n1b[i], f1w[i], f1b[i],
                  f2w[i], f2b[i], n2g[i], n2b[i])
    return _final(x, ng, nb, proj_w, proj_b)
